# Initial kernel scaffold; baseline (speedup 1.0000x reference)
import jax
import jax.numpy as jnp
from jax.experimental import pallas as pl


def kernel(state, Wi1, bi1, Wi2, bi2, Wb1, bb1, Wb2, bb2, action):
    raise NotImplementedError("write your pallas kernel here")


# speedup vs baseline: 1.1799x; 1.1799x over previous
import dataclasses
import functools

import jax
import jax.numpy as jnp
from jax import lax
from jax.experimental import pallas as pl
from jax.experimental.pallas import tpu as pltpu
from jax.experimental.pallas import tpu_sc as plsc

_B, _N, _D = 128, 32768, 32
_NEG = float(jnp.finfo(jnp.float32).min)
_NC, _NS = 2, 16
_NW = _NC * _NS
_RPW = _B // _NW
_R = 256


def _sc_gather_body(fc_hbm, idx_hbm, fci_hbm, fc_v, idx_v, out_v, sem_a, sem_b):
    cid = lax.axis_index("c")
    sid = lax.axis_index("s")
    wid = sid * _NC + cid

    @pl.loop(0, _RPW)
    def _row(r):
        b = wid * _RPW + r
        cp_fc = pltpu.async_copy(fc_hbm.at[b], fc_v, sem_a)
        cp_idx = pltpu.async_copy(idx_hbm.at[b], idx_v, sem_b)
        cp_fc.wait()
        cp_idx.wait()

        @pl.loop(0, _N, step=16)
        def _grp(i):
            ivec = idx_v[pl.ds(i, 16)]
            out_v[pl.ds(i, 16)] = plsc.load_gather(fc_v, [ivec])

        pltpu.sync_copy(out_v, fci_hbm.at[b])


def _sc_compiler_params():
    cp = pltpu.CompilerParams()
    if "needs_layout_passes" in pltpu.CompilerParams.__dataclass_fields__:
        cp = dataclasses.replace(cp, needs_layout_passes=False)
    return cp


def _sc_gather(fc, idx):
    mesh = plsc.VectorSubcoreMesh(core_axis_name="c", subcore_axis_name="s")
    return pl.kernel(
        _sc_gather_body,
        out_type=jax.ShapeDtypeStruct((_B, _N), jnp.float32),
        mesh=mesh,
        scratch_types=[
            pltpu.VMEM((_N,), jnp.float32),
            pltpu.VMEM((_N,), jnp.int32),
            pltpu.VMEM((_N,), jnp.float32),
            pltpu.SemaphoreType.DMA,
            pltpu.SemaphoreType.DMA,
        ],
        compiler_params=_sc_compiler_params(),
    )(fc, idx)


def _tc_body(action_ref, iw_ref, fc_ref, temp_ref, fci_ref,
             wi1_ref, bi1_ref, wi2_ref, bi2_ref,
             wb1_ref, bb1_ref, wb2_ref, bb2_ref, out_ref):
    item = action_ref[0, 0, 0]
    bin_ = action_ref[0, 0, 1]
    iw = iw_ref[0]
    fc = fc_ref[0]
    temp = temp_ref[0]
    fci = fci_ref[0]

    rid = lax.broadcasted_iota(jnp.int32, (_R, 128), 0)
    lid = lax.broadcasted_iota(jnp.int32, (_R, 128), 1)
    nid = rid * 128 + lid
    item_mask = nid == item
    bin_mask = nid == bin_
    zero = jnp.zeros((_R, 128), jnp.float32)

    iw_b = jnp.sum(jnp.where(item_mask, iw, zero))

    acc_i = jnp.full((_R, 128), bi2_ref[0], jnp.float32)
    for d in range(_D):
        h = jnp.maximum(
            iw * wi1_ref[d, 0] + fci * wi1_ref[d, 1] + temp * wi1_ref[d, 2]
            + bi1_ref[d], 0.0)
        acc_i = acc_i + h * wi2_ref[0, d]

    acc_b = jnp.full((_R, 128), bb2_ref[0], jnp.float32)
    for d in range(_D):
        h = jnp.maximum(
            fc * wb1_ref[d, 1] + temp * wb1_ref[d, 2]
            + (bb1_ref[d] + iw_b * wb1_ref[d, 0]), 0.0)
        acc_b = acc_b + h * wb2_ref[0, d]
    masked = (iw_b - fc > 0.0) | item_mask
    acc_b = acc_b + jnp.where(masked, _NEG, zero)

    pick_i = jnp.sum(jnp.where(item_mask, acc_i, zero))
    pick_b = jnp.sum(jnp.where(bin_mask, acc_b, zero))
    lse_i = jnp.log(jnp.sum(jnp.exp(acc_i)))
    lse_b = jnp.log(jnp.sum(jnp.exp(acc_b)))
    out_ref[0, 0, 0] = (pick_i - lse_i) + (pick_b - lse_b)


def _tc_main(action, iw, fc, temp, fci, Wi1, bi1, Wi2, bi2, Wb1, bb1, Wb2, bb2):
    plane = pl.BlockSpec((1, _R, 128), lambda b: (b, 0, 0))

    def smem(shape):
        return pl.BlockSpec(shape, lambda b: tuple(0 for _ in shape),
                            memory_space=pltpu.SMEM)

    return pl.pallas_call(
        _tc_body,
        grid=(_B,),
        in_specs=[
            pl.BlockSpec((1, 1, 2), lambda b: (b, 0, 0),
                         memory_space=pltpu.SMEM),
            plane, plane, plane, plane,
            smem((_D, 3)), smem((_D,)), smem((1, _D)), smem((1,)),
            smem((_D, 3)), smem((_D,)), smem((1, _D)), smem((1,)),
        ],
        out_specs=pl.BlockSpec((1, 1, 1), lambda b: (b, 0, 0),
                               memory_space=pltpu.SMEM),
        out_shape=jax.ShapeDtypeStruct((_B, 1, 1), jnp.float32),
        compiler_params=pltpu.CompilerParams(
            dimension_semantics=("arbitrary",)),
    )(action.reshape(_B, 1, 2), iw, fc, temp, fci,
      Wi1, bi1, Wi2, bi2, Wb1, bb1, Wb2, bb2)


def kernel(state, Wi1, bi1, Wi2, bi2, Wb1, bb1, Wb2, bb2, action):
    idx = state[..., 0].astype(jnp.int32)
    iw = state[..., 1]
    fc = state[..., 2]
    temp = state[..., 3]
    fci = _sc_gather(fc, idx)
    r3 = lambda a: a.reshape(_B, _R, 128)
    out = _tc_main(action, r3(iw), r3(fc), r3(temp), r3(fci),
                   Wi1, bi1, Wi2, bi2, Wb1, bb1, Wb2, bb2)
    return out.reshape(_B)
